# Initial kernel scaffold; baseline (speedup 1.0000x reference)
import jax
import jax.numpy as jnp
from jax.experimental import pallas as pl


def kernel(x, edge_index, W_gat, att_src, att_dst, b_gat, W1, b1, W2, b2):
    raise NotImplementedError("write your pallas kernel here")


# speedup vs baseline: 33.8642x; 33.8642x over previous
import functools

import jax
import jax.numpy as jnp
from jax import lax
from jax.experimental import pallas as pl
from jax.experimental.pallas import tpu as pltpu
from jax.experimental.pallas import tpu_sc as plsc

N = 10000
E = 320000
IN = 128
H = 8
C = 32
HID = H * C
OUT = 256
HH = HID // 2

NS = 16
K = 128
EPAD = ((E + NS * K - 1) // (NS * K)) * (NS * K)
EPT = EPAD // NS
NCH = EPT // K
ROWS_PT = 632
NROW = NS * ROWS_PT

BN = 1000


def _pre_body(x_ref, wg_ref, asrc_ref, adst_ref, eexp_ref,
              h_ref, s_ref, d_ref, exs_ref, acc0_ref):
    h = jnp.dot(x_ref[...], wg_ref[...], preferred_element_type=jnp.float32)
    s = jnp.dot(h, asrc_ref[...], preferred_element_type=jnp.float32)
    d = jnp.dot(h, adst_ref[...], preferred_element_type=jnp.float32)
    ss = s + d
    exs = jnp.exp(jnp.where(ss > 0, ss, 0.2 * ss))
    h_ref[...] = h
    s_ref[...] = s
    d_ref[...] = d
    exs_ref[...] = exs
    acc0_ref[...] = h * jnp.dot(exs, eexp_ref[...],
                                preferred_element_type=jnp.float32)


def _post_body(acc_ref, den_ref, eexp_ref, bg_ref, w1_ref, b1_ref,
               w2_ref, b2_ref, z_ref, z1_ref, z2_ref):
    den = jnp.dot(den_ref[...], eexp_ref[...],
                  preferred_element_type=jnp.float32)
    z = acc_ref[...] / (den + 1e-16) + bg_ref[...]
    z_ref[...] = z
    z1_ref[...] = jnp.dot(z, w1_ref[...],
                          preferred_element_type=jnp.float32) + b1_ref[...]
    z2_ref[...] = jnp.dot(z, w2_ref[...],
                          preferred_element_type=jnp.float32) + b2_ref[...]


WROW = HH + 16


def _edge_body(src_hbm, dst_hbm, d_hbm, h2_hbm, acc0_hbm,
               acc_out,
               src_v, dst_v, idx2_v, idxd_v, drows_v, hrows_v,
               acc_sh, sem_d, sem_h):
    c = lax.axis_index("c")
    s = lax.axis_index("s")
    coff = c * NROW
    r0 = s * ROWS_PT
    pltpu.sync_copy(acc0_hbm.at[pl.ds(coff + r0, ROWS_PT)],
                    acc_sh.at[pl.ds(r0, ROWS_PT)])
    plsc.subcore_barrier()

    base = s * EPT
    pat = lax.iota(jnp.int32, 16) & 3

    def chunk_body(ch, carry):
        off = base + ch * K
        pltpu.sync_copy(src_hbm.at[pl.ds(off, K)], src_v)
        pltpu.sync_copy(dst_hbm.at[pl.ds(off, K)], dst_v)

        def addoff(i, carry2):
            idx2_v[pl.ds(i * 16, 16)] = src_v[pl.ds(i * 16, 16)] + coff
            idxd_v[pl.ds(i * 16, 16)] = dst_v[pl.ds(i * 16, 16)] + coff
            return carry2
        lax.fori_loop(0, K // 16, addoff, 0)

        cp_d = pltpu.async_copy(d_hbm.at[idxd_v], drows_v, sem_d)
        cp_h = pltpu.async_copy(h2_hbm.at[idx2_v], hrows_v, sem_h)
        cp_d.wait()
        cp_h.wait()

        def edge_body(e, carry2):
            sv = hrows_v[e, pl.ds(HH, 16)]
            dv = drows_v[e, pl.ds(0, 16)]
            al = sv + dv
            ex = jnp.exp(jnp.where(al > 0, al, 0.2 * al))
            m0 = jnp.full((16,), ex[0])
            m1 = jnp.full((16,), ex[1])
            m2 = jnp.full((16,), ex[2])
            m3 = jnp.full((16,), ex[3])
            ms = (m0, m1, m2, m3)
            for j in range(HH // 16):
                hrows_v[e, pl.ds(j * 16, 16)] = (
                    hrows_v[e, pl.ds(j * 16, 16)] * ms[j // 2])
            mrep = jnp.where(pat == 0, m0,
                             jnp.where(pat == 1, m1,
                                       jnp.where(pat == 2, m2, m3)))
            hrows_v[e, pl.ds(HH, 16)] = sv * mrep
            return carry2
        lax.fori_loop(0, K, edge_body, 0)

        pltpu.sync_copy(hrows_v, acc_sh.at[dst_v], add=True)
        return carry
    lax.fori_loop(0, NCH, chunk_body, 0)

    plsc.subcore_barrier()
    pltpu.sync_copy(acc_sh.at[pl.ds(r0, ROWS_PT)],
                    acc_out.at[pl.ds(coff + r0, ROWS_PT)])


def _edge_pass(srcp, dstp, D16, h2, acc0):
    mesh = plsc.VectorSubcoreMesh(core_axis_name="c", subcore_axis_name="s")
    f = functools.partial(
        pl.kernel,
        mesh=mesh,
        compiler_params=pltpu.CompilerParams(use_tc_tiling_on_sc=False),
        out_type=jax.ShapeDtypeStruct((2 * NROW, WROW), jnp.float32),
        scratch_types=[
            pltpu.VMEM((K,), jnp.int32),
            pltpu.VMEM((K,), jnp.int32),
            pltpu.VMEM((K,), jnp.int32),
            pltpu.VMEM((K,), jnp.int32),
            pltpu.VMEM((K, 16), jnp.float32),
            pltpu.VMEM((K, WROW), jnp.float32),
            pltpu.VMEM_SHARED((NROW, WROW), jnp.float32),
            pltpu.SemaphoreType.DMA,
            pltpu.SemaphoreType.DMA,
        ],
    )(_edge_body)
    return f(srcp, dstp, D16, h2, acc0)


def kernel(x, edge_index, W_gat, att_src, att_dst, b_gat, W1, b1, W2, b2):
    f32 = jnp.float32
    Asrc = (jnp.eye(H, dtype=f32)[:, None, :] * att_src[:, :, None]
            ).reshape(HID, H)
    Adst = (jnp.eye(H, dtype=f32)[:, None, :] * att_dst[:, :, None]
            ).reshape(HID, H)
    E_exp = jnp.repeat(jnp.eye(H, dtype=f32), C, axis=1)

    h, S, D, exs, acc0f = pl.pallas_call(
        _pre_body,
        grid=(N // BN,),
        in_specs=[
            pl.BlockSpec((BN, IN), lambda i: (i, 0)),
            pl.BlockSpec((IN, HID), lambda i: (0, 0)),
            pl.BlockSpec((HID, H), lambda i: (0, 0)),
            pl.BlockSpec((HID, H), lambda i: (0, 0)),
            pl.BlockSpec((H, HID), lambda i: (0, 0)),
        ],
        out_specs=[
            pl.BlockSpec((BN, HID), lambda i: (i, 0)),
            pl.BlockSpec((BN, H), lambda i: (i, 0)),
            pl.BlockSpec((BN, H), lambda i: (i, 0)),
            pl.BlockSpec((BN, H), lambda i: (i, 0)),
            pl.BlockSpec((BN, HID), lambda i: (i, 0)),
        ],
        out_shape=[
            jax.ShapeDtypeStruct((N, HID), f32),
            jax.ShapeDtypeStruct((N, H), f32),
            jax.ShapeDtypeStruct((N, H), f32),
            jax.ShapeDtypeStruct((N, H), f32),
            jax.ShapeDtypeStruct((N, HID), f32),
        ],
    )(x, W_gat, Asrc, Adst, E_exp)

    ones4 = jnp.ones((N, 4), f32)
    zero8 = jnp.zeros((N, 8), f32)
    zrow_w = jnp.zeros((NROW - N, WROW), f32)
    zrow_16 = jnp.zeros((NROW - N, 16), f32)
    h2 = jnp.concatenate([
        jnp.concatenate([h[:, :HH], S[:, :4], ones4, zero8], axis=1),
        zrow_w,
        jnp.concatenate([h[:, HH:], S[:, 4:], ones4, zero8], axis=1),
        zrow_w,
    ], axis=0)
    D16 = jnp.concatenate([
        jnp.concatenate([D[:, :4], jnp.zeros((N, 12), f32)], axis=1),
        zrow_16,
        jnp.concatenate([D[:, 4:], jnp.zeros((N, 12), f32)], axis=1),
        zrow_16,
    ], axis=0)
    acc0 = jnp.concatenate([
        jnp.concatenate([acc0f[:, :HH], jnp.zeros((N, 4), f32),
                         exs[:, :4], zero8], axis=1),
        zrow_w,
        jnp.concatenate([acc0f[:, HH:], jnp.zeros((N, 4), f32),
                         exs[:, 4:], zero8], axis=1),
        zrow_w,
    ], axis=0)

    src = edge_index[0]
    dst = edge_index[1]
    pad = EPAD - E
    srcp = jnp.concatenate([src, jnp.zeros((pad,), jnp.int32)])
    dstp = jnp.concatenate([dst, jnp.full((pad,), N, jnp.int32)])

    acc = _edge_pass(srcp, dstp, D16, h2, acc0)

    accF = jnp.concatenate(
        [acc[:N, :HH], acc[NROW:NROW + N, :HH]], axis=1)
    denF = jnp.concatenate(
        [acc[:N, HH + 4:HH + 8], acc[NROW:NROW + N, HH + 4:HH + 8]],
        axis=1)

    z, z1, z2 = pl.pallas_call(
        _post_body,
        grid=(N // BN,),
        in_specs=[
            pl.BlockSpec((BN, HID), lambda i: (i, 0)),
            pl.BlockSpec((BN, H), lambda i: (i, 0)),
            pl.BlockSpec((H, HID), lambda i: (0, 0)),
            pl.BlockSpec((1, HID), lambda i: (0, 0)),
            pl.BlockSpec((HID, OUT), lambda i: (0, 0)),
            pl.BlockSpec((1, OUT), lambda i: (0, 0)),
            pl.BlockSpec((HID, OUT), lambda i: (0, 0)),
            pl.BlockSpec((1, OUT), lambda i: (0, 0)),
        ],
        out_specs=[
            pl.BlockSpec((BN, HID), lambda i: (i, 0)),
            pl.BlockSpec((BN, OUT), lambda i: (i, 0)),
            pl.BlockSpec((BN, OUT), lambda i: (i, 0)),
        ],
        out_shape=[
            jax.ShapeDtypeStruct((N, HID), f32),
            jax.ShapeDtypeStruct((N, OUT), f32),
            jax.ShapeDtypeStruct((N, OUT), f32),
        ],
    )(accF, denF, E_exp, b_gat.reshape(1, HID), W1, b1.reshape(1, OUT),
      W2, b2.reshape(1, OUT))
    return (z, z1, z2)


# speedup vs baseline: 49.7393x; 1.4688x over previous
import functools

import jax
import jax.numpy as jnp
from jax import lax
from jax.experimental import pallas as pl
from jax.experimental.pallas import tpu as pltpu
from jax.experimental.pallas import tpu_sc as plsc

N = 10000
E = 320000
IN = 128
H = 8
C = 32
HID = H * C
OUT = 256
HH = HID // 2

NS = 16
K = 112
NCH = 180
EPT = NCH * K
EPAD = NS * EPT
ROWS_PT = 632
NROW = NS * ROWS_PT

BN = 1000


def _pre_body(x_ref, wg_ref, asrc_ref, adst_ref, eexp_ref,
              h_ref, s_ref, d_ref, exs_ref, acc0_ref):
    h = jnp.dot(x_ref[...], wg_ref[...], preferred_element_type=jnp.float32)
    s = jnp.dot(h, asrc_ref[...], preferred_element_type=jnp.float32)
    d = jnp.dot(h, adst_ref[...], preferred_element_type=jnp.float32)
    ss = s + d
    exs = jnp.exp(jnp.where(ss > 0, ss, 0.2 * ss))
    h_ref[...] = h
    s_ref[...] = s
    d_ref[...] = d
    exs_ref[...] = exs
    acc0_ref[...] = h * jnp.dot(exs, eexp_ref[...],
                                preferred_element_type=jnp.float32)


def _post_body(acc_ref, den_ref, eexp_ref, bg_ref, w1_ref, b1_ref,
               w2_ref, b2_ref, z_ref, z1_ref, z2_ref):
    den = jnp.dot(den_ref[...], eexp_ref[...],
                  preferred_element_type=jnp.float32)
    z = acc_ref[...] / (den + 1e-16) + bg_ref[...]
    z_ref[...] = z
    z1_ref[...] = jnp.dot(z, w1_ref[...],
                          preferred_element_type=jnp.float32) + b1_ref[...]
    z2_ref[...] = jnp.dot(z, w2_ref[...],
                          preferred_element_type=jnp.float32) + b2_ref[...]


WROW = HH + 16


def _edge_body(src_hbm, dst_hbm, d_hbm, h2_hbm, acc0_hbm,
               acc_out,
               src_v, dst_v, idx2_v, idxd_v, dsc_v, drows_v, hrows_v,
               acc_sh, sem_is, sem_id, sem_d, sem_h, sem_s):
    c = lax.axis_index("c")
    s = lax.axis_index("s")
    coff = c * NROW
    r0 = s * ROWS_PT
    pltpu.sync_copy(acc0_hbm.at[pl.ds(coff + r0, ROWS_PT)],
                    acc_sh.at[pl.ds(r0, ROWS_PT)])
    plsc.subcore_barrier()

    base = s * EPT

    def issue_idx(ch, b):
        off = base + ch * K
        pltpu.async_copy(src_hbm.at[pl.ds(off, K)], src_v.at[b],
                         sem_is.at[b])
        pltpu.async_copy(dst_hbm.at[pl.ds(off, K)], dst_v.at[b],
                         sem_id.at[b])

    def wait_idx(b):
        pltpu.make_async_copy(src_hbm.at[pl.ds(0, K)], src_v.at[b],
                              sem_is.at[b]).wait()
        pltpu.make_async_copy(dst_hbm.at[pl.ds(0, K)], dst_v.at[b],
                              sem_id.at[b]).wait()

    def comp_idx(b):
        def f(i, carry):
            sl = pl.ds(i * 16, 16)
            idx2_v[b, sl] = src_v[b, sl] + coff
            idxd_v[b, sl] = dst_v[b, sl] + coff
            dsc_v[b, sl] = dst_v[b, sl]
            return carry
        lax.fori_loop(0, K // 16, f, 0)

    def issue_gathers(b):
        pltpu.async_copy(d_hbm.at[idxd_v.at[b]], drows_v.at[b], sem_d.at[b])
        pltpu.async_copy(h2_hbm.at[idx2_v.at[b]], hrows_v.at[b], sem_h.at[b])

    def wait_gathers(b):
        pltpu.make_async_copy(d_hbm.at[idxd_v.at[b]], drows_v.at[b],
                              sem_d.at[b]).wait()
        pltpu.make_async_copy(h2_hbm.at[idx2_v.at[b]], hrows_v.at[b],
                              sem_h.at[b]).wait()

    def issue_scatter(b):
        pltpu.async_copy(hrows_v.at[b], acc_sh.at[dsc_v.at[b]], sem_s.at[b],
                         add=True)

    def wait_scatter(b):
        pltpu.make_async_copy(hrows_v.at[b], acc_sh.at[dsc_v.at[b]],
                              sem_s.at[b]).wait()

    def compute(b):
        def edge_f(e, carry):
            sv = hrows_v[b, e, pl.ds(HH, 16)]
            dv = drows_v[b, e, pl.ds(0, 16)]
            al = sv + dv
            ex = jnp.exp(jnp.where(al > 0, al, 0.2 * al))
            for j in range(HH // 16):
                hrows_v[b, e, pl.ds(j * 16, 16)] = (
                    hrows_v[b, e, pl.ds(j * 16, 16)]
                    * jnp.full((16,), ex[j // 2]))
            hrows_v[b, e, pl.ds(HH, 16)] = ex
            return carry
        lax.fori_loop(0, K, edge_f, 0)

    def step(ch, b):
        b2 = 1 - b

        @pl.when(ch < NCH - 1)
        def _():
            wait_idx(b2)

        @pl.when(ch >= 1)
        def _():
            wait_scatter(b2)

        @pl.when(ch < NCH - 1)
        def _():
            comp_idx(b2)
            issue_gathers(b2)

        @pl.when(ch < NCH - 2)
        def _():
            issue_idx(ch + 2, b)

        wait_gathers(b)
        compute(b)
        issue_scatter(b)

    issue_idx(0, 0)
    issue_idx(1, 1)
    wait_idx(0)
    comp_idx(0)
    issue_gathers(0)

    def pair(g, carry):
        step(2 * g, 0)
        step(2 * g + 1, 1)
        return carry
    lax.fori_loop(0, NCH // 2, pair, 0)
    wait_scatter(1)

    plsc.subcore_barrier()
    pltpu.sync_copy(acc_sh.at[pl.ds(r0, ROWS_PT)],
                    acc_out.at[pl.ds(coff + r0, ROWS_PT)])


def _edge_pass(srcp, dstp, D16, h2, acc0):
    mesh = plsc.VectorSubcoreMesh(core_axis_name="c", subcore_axis_name="s")
    f = functools.partial(
        pl.kernel,
        mesh=mesh,
        compiler_params=pltpu.CompilerParams(use_tc_tiling_on_sc=False),
        out_type=jax.ShapeDtypeStruct((2 * NROW, WROW), jnp.float32),
        scratch_types=[
            pltpu.VMEM((2, K), jnp.int32),
            pltpu.VMEM((2, K), jnp.int32),
            pltpu.VMEM((2, K), jnp.int32),
            pltpu.VMEM((2, K), jnp.int32),
            pltpu.VMEM((2, K), jnp.int32),
            pltpu.VMEM((2, K, 16), jnp.float32),
            pltpu.VMEM((2, K, WROW), jnp.float32),
            pltpu.VMEM_SHARED((NROW, WROW), jnp.float32),
            pltpu.SemaphoreType.DMA((2,)),
            pltpu.SemaphoreType.DMA((2,)),
            pltpu.SemaphoreType.DMA((2,)),
            pltpu.SemaphoreType.DMA((2,)),
            pltpu.SemaphoreType.DMA((2,)),
        ],
    )(_edge_body)
    return f(srcp, dstp, D16, h2, acc0)


def kernel(x, edge_index, W_gat, att_src, att_dst, b_gat, W1, b1, W2, b2):
    f32 = jnp.float32
    Asrc = (jnp.eye(H, dtype=f32)[:, None, :] * att_src[:, :, None]
            ).reshape(HID, H)
    Adst = (jnp.eye(H, dtype=f32)[:, None, :] * att_dst[:, :, None]
            ).reshape(HID, H)
    E_exp = jnp.repeat(jnp.eye(H, dtype=f32), C, axis=1)

    h, S, D, exs, acc0f = pl.pallas_call(
        _pre_body,
        grid=(N // BN,),
        in_specs=[
            pl.BlockSpec((BN, IN), lambda i: (i, 0)),
            pl.BlockSpec((IN, HID), lambda i: (0, 0)),
            pl.BlockSpec((HID, H), lambda i: (0, 0)),
            pl.BlockSpec((HID, H), lambda i: (0, 0)),
            pl.BlockSpec((H, HID), lambda i: (0, 0)),
        ],
        out_specs=[
            pl.BlockSpec((BN, HID), lambda i: (i, 0)),
            pl.BlockSpec((BN, H), lambda i: (i, 0)),
            pl.BlockSpec((BN, H), lambda i: (i, 0)),
            pl.BlockSpec((BN, H), lambda i: (i, 0)),
            pl.BlockSpec((BN, HID), lambda i: (i, 0)),
        ],
        out_shape=[
            jax.ShapeDtypeStruct((N, HID), f32),
            jax.ShapeDtypeStruct((N, H), f32),
            jax.ShapeDtypeStruct((N, H), f32),
            jax.ShapeDtypeStruct((N, H), f32),
            jax.ShapeDtypeStruct((N, HID), f32),
        ],
    )(x, W_gat, Asrc, Adst, E_exp)

    zrow_w = jnp.zeros((NROW - N, WROW), f32)
    zrow_16 = jnp.zeros((NROW - N, 16), f32)
    h2 = jnp.concatenate([
        jnp.concatenate([h[:, :HH], jnp.tile(S[:, :4], (1, 4))], axis=1),
        zrow_w,
        jnp.concatenate([h[:, HH:], jnp.tile(S[:, 4:], (1, 4))], axis=1),
        zrow_w,
    ], axis=0)
    D16 = jnp.concatenate([
        jnp.tile(D[:, :4], (1, 4)),
        zrow_16,
        jnp.tile(D[:, 4:], (1, 4)),
        zrow_16,
    ], axis=0)
    acc0 = jnp.concatenate([
        jnp.concatenate([acc0f[:, :HH], jnp.tile(exs[:, :4], (1, 4))],
                        axis=1),
        zrow_w,
        jnp.concatenate([acc0f[:, HH:], jnp.tile(exs[:, 4:], (1, 4))],
                        axis=1),
        zrow_w,
    ], axis=0)

    src = edge_index[0]
    dst = edge_index[1]
    pad = EPAD - E
    srcp = jnp.concatenate([src, jnp.zeros((pad,), jnp.int32)])
    dstp = jnp.concatenate([dst, jnp.full((pad,), N, jnp.int32)])

    acc = _edge_pass(srcp, dstp, D16, h2, acc0)

    accF = jnp.concatenate(
        [acc[:N, :HH], acc[NROW:NROW + N, :HH]], axis=1)
    denF = jnp.concatenate(
        [acc[:N, HH:HH + 4], acc[NROW:NROW + N, HH:HH + 4]],
        axis=1)

    z, z1, z2 = pl.pallas_call(
        _post_body,
        grid=(N // BN,),
        in_specs=[
            pl.BlockSpec((BN, HID), lambda i: (i, 0)),
            pl.BlockSpec((BN, H), lambda i: (i, 0)),
            pl.BlockSpec((H, HID), lambda i: (0, 0)),
            pl.BlockSpec((1, HID), lambda i: (0, 0)),
            pl.BlockSpec((HID, OUT), lambda i: (0, 0)),
            pl.BlockSpec((1, OUT), lambda i: (0, 0)),
            pl.BlockSpec((HID, OUT), lambda i: (0, 0)),
            pl.BlockSpec((1, OUT), lambda i: (0, 0)),
        ],
        out_specs=[
            pl.BlockSpec((BN, HID), lambda i: (i, 0)),
            pl.BlockSpec((BN, OUT), lambda i: (i, 0)),
            pl.BlockSpec((BN, OUT), lambda i: (i, 0)),
        ],
        out_shape=[
            jax.ShapeDtypeStruct((N, HID), f32),
            jax.ShapeDtypeStruct((N, OUT), f32),
            jax.ShapeDtypeStruct((N, OUT), f32),
        ],
    )(accF, denF, E_exp, b_gat.reshape(1, HID), W1, b1.reshape(1, OUT),
      W2, b2.reshape(1, OUT))
    return (z, z1, z2)


# speedup vs baseline: 55.9853x; 1.1256x over previous
import functools

import jax
import jax.numpy as jnp
from jax import lax
from jax.experimental import pallas as pl
from jax.experimental.pallas import tpu as pltpu
from jax.experimental.pallas import tpu_sc as plsc

N = 10000
E = 320000
IN = 128
H = 8
C = 32
HID = H * C
OUT = 256
HH = HID // 2

NS = 16
K = 112
NCH = 180
EPT = NCH * K
EPAD = NS * EPT
ROWS_PT = 632
NROW = NS * ROWS_PT

BN = 1000


def _pre_body(x_ref, wg_ref, asrc_ref, adst_ref, e4_ref,
              h2_ref, d16_ref, acc0_ref):
    hh = jnp.dot(x_ref[...], wg_ref[...], preferred_element_type=jnp.float32)
    s4 = jnp.dot(hh, asrc_ref[0], preferred_element_type=jnp.float32)
    d4 = jnp.dot(hh, adst_ref[0], preferred_element_type=jnp.float32)
    ss = s4 + d4
    exs4 = jnp.exp(jnp.where(ss > 0, ss, 0.2 * ss))
    h2_ref[...] = jnp.concatenate([hh, s4, s4, s4, s4], axis=1)
    d16_ref[...] = jnp.concatenate([d4, d4, d4, d4], axis=1)
    acc0_ref[...] = jnp.concatenate(
        [hh * jnp.dot(exs4, e4_ref[...], preferred_element_type=jnp.float32),
         exs4, exs4, exs4, exs4], axis=1)


def _post_body(a0_ref, a1_ref, p_ref, g_ref, bg_ref,
               w1_ref, b1_ref, w2_ref, b2_ref, z_ref, z1_ref, z2_ref):
    a0 = a0_ref[...]
    a1 = a1_ref[...]
    p = p_ref[...]
    g = g_ref[...]
    den0 = jnp.dot(a0, g, preferred_element_type=jnp.float32)
    den1 = jnp.dot(a1, g, preferred_element_type=jnp.float32)
    m0 = jnp.dot(a0, p, preferred_element_type=jnp.float32)
    m1 = jnp.dot(a1, p, preferred_element_type=jnp.float32)
    z = jnp.concatenate(
        [m0 / (den0 + 1e-16), m1 / (den1 + 1e-16)],
        axis=1) + bg_ref[...]
    z_ref[...] = z
    z1_ref[...] = jnp.dot(z, w1_ref[...],
                          preferred_element_type=jnp.float32) + b1_ref[...]
    z2_ref[...] = jnp.dot(z, w2_ref[...],
                          preferred_element_type=jnp.float32) + b2_ref[...]


WROW = HH + 16


def _edge_body(src_hbm, dst_hbm, d_hbm, h2_hbm, acc0_hbm,
               acc_out,
               src_v, dst_v, idx2_v, idxd_v, dsc_v, drows_v, hrows_v,
               acc_sh, sem_is, sem_id, sem_d, sem_h, sem_s):
    c = lax.axis_index("c")
    s = lax.axis_index("s")
    coff = c * NROW
    r0 = s * ROWS_PT
    pltpu.sync_copy(acc0_hbm.at[pl.ds(coff + r0, ROWS_PT)],
                    acc_sh.at[pl.ds(r0, ROWS_PT)])
    plsc.subcore_barrier()

    base = s * EPT

    def issue_idx(ch, b):
        off = base + ch * K
        pltpu.async_copy(src_hbm.at[pl.ds(off, K)], src_v.at[b],
                         sem_is.at[b])
        pltpu.async_copy(dst_hbm.at[pl.ds(off, K)], dst_v.at[b],
                         sem_id.at[b])

    def wait_idx(b):
        pltpu.make_async_copy(src_hbm.at[pl.ds(0, K)], src_v.at[b],
                              sem_is.at[b]).wait()
        pltpu.make_async_copy(dst_hbm.at[pl.ds(0, K)], dst_v.at[b],
                              sem_id.at[b]).wait()

    def comp_idx(b):
        def f(i, carry):
            sl = pl.ds(i * 16, 16)
            idx2_v[b, sl] = src_v[b, sl] + coff
            idxd_v[b, sl] = dst_v[b, sl] + coff
            dsc_v[b, sl] = dst_v[b, sl]
            return carry
        lax.fori_loop(0, K // 16, f, 0)

    def issue_gathers(b):
        pltpu.async_copy(d_hbm.at[idxd_v.at[b]], drows_v.at[b], sem_d.at[b])
        pltpu.async_copy(h2_hbm.at[idx2_v.at[b]], hrows_v.at[b], sem_h.at[b])

    def wait_gathers(b):
        pltpu.make_async_copy(d_hbm.at[idxd_v.at[b]], drows_v.at[b],
                              sem_d.at[b]).wait()
        pltpu.make_async_copy(h2_hbm.at[idx2_v.at[b]], hrows_v.at[b],
                              sem_h.at[b]).wait()

    def issue_scatter(b):
        pltpu.async_copy(hrows_v.at[b], acc_sh.at[dsc_v.at[b]], sem_s.at[b],
                         add=True)

    def wait_scatter(b):
        pltpu.make_async_copy(hrows_v.at[b], acc_sh.at[dsc_v.at[b]],
                              sem_s.at[b]).wait()

    def compute(b):
        def edge_f(e, carry):
            sv = hrows_v[b, e, pl.ds(HH, 16)]
            dv = drows_v[b, e, pl.ds(0, 16)]
            al = sv + dv
            ex = jnp.exp(jnp.where(al > 0, al, 0.2 * al))
            for j in range(HH // 16):
                hrows_v[b, e, pl.ds(j * 16, 16)] = (
                    hrows_v[b, e, pl.ds(j * 16, 16)]
                    * jnp.full((16,), ex[j // 2]))
            hrows_v[b, e, pl.ds(HH, 16)] = ex
            return carry
        lax.fori_loop(0, K, edge_f, 0)

    def step(ch, b):
        b2 = 1 - b

        @pl.when(ch < NCH - 1)
        def _():
            wait_idx(b2)

        @pl.when(ch >= 1)
        def _():
            wait_scatter(b2)

        @pl.when(ch < NCH - 1)
        def _():
            comp_idx(b2)
            issue_gathers(b2)

        @pl.when(ch < NCH - 2)
        def _():
            issue_idx(ch + 2, b)

        wait_gathers(b)
        compute(b)
        issue_scatter(b)

    issue_idx(0, 0)
    issue_idx(1, 1)
    wait_idx(0)
    comp_idx(0)
    issue_gathers(0)

    def pair(g, carry):
        step(2 * g, 0)
        step(2 * g + 1, 1)
        return carry
    lax.fori_loop(0, NCH // 2, pair, 0)
    wait_scatter(1)

    plsc.subcore_barrier()
    pltpu.sync_copy(acc_sh.at[pl.ds(r0, ROWS_PT)],
                    acc_out.at[pl.ds(coff + r0, ROWS_PT)])


def _edge_pass(srcp, dstp, D16, h2, acc0):
    mesh = plsc.VectorSubcoreMesh(core_axis_name="c", subcore_axis_name="s")
    f = functools.partial(
        pl.kernel,
        mesh=mesh,
        compiler_params=pltpu.CompilerParams(use_tc_tiling_on_sc=False),
        out_type=jax.ShapeDtypeStruct((2 * NROW, WROW), jnp.float32),
        scratch_types=[
            pltpu.VMEM((2, K), jnp.int32),
            pltpu.VMEM((2, K), jnp.int32),
            pltpu.VMEM((2, K), jnp.int32),
            pltpu.VMEM((2, K), jnp.int32),
            pltpu.VMEM((2, K), jnp.int32),
            pltpu.VMEM((2, K, 16), jnp.float32),
            pltpu.VMEM((2, K, WROW), jnp.float32),
            pltpu.VMEM_SHARED((NROW, WROW), jnp.float32),
            pltpu.SemaphoreType.DMA((2,)),
            pltpu.SemaphoreType.DMA((2,)),
            pltpu.SemaphoreType.DMA((2,)),
            pltpu.SemaphoreType.DMA((2,)),
            pltpu.SemaphoreType.DMA((2,)),
        ],
    )(_edge_body)
    return f(srcp, dstp, D16, h2, acc0)


def kernel(x, edge_index, W_gat, att_src, att_dst, b_gat, W1, b1, W2, b2):
    f32 = jnp.float32
    Asrc = (jnp.eye(H, dtype=f32)[:, None, :] * att_src[:, :, None]
            ).reshape(HID, H)
    Adst = (jnp.eye(H, dtype=f32)[:, None, :] * att_dst[:, :, None]
            ).reshape(HID, H)
    E4 = jnp.repeat(jnp.eye(4, dtype=f32), C, axis=1)

    h2, D16, acc0 = pl.pallas_call(
        _pre_body,
        grid=(2, NS),
        in_specs=[
            pl.BlockSpec((ROWS_PT, IN), lambda c, i: (i, 0)),
            pl.BlockSpec((IN, HH), lambda c, i: (0, c)),
            pl.BlockSpec((1, HH, 4), lambda c, i: (c, 0, 0)),
            pl.BlockSpec((1, HH, 4), lambda c, i: (c, 0, 0)),
            pl.BlockSpec((4, HH), lambda c, i: (0, 0)),
        ],
        out_specs=[
            pl.BlockSpec((ROWS_PT, WROW), lambda c, i: (c * NS + i, 0)),
            pl.BlockSpec((ROWS_PT, 16), lambda c, i: (c * NS + i, 0)),
            pl.BlockSpec((ROWS_PT, WROW), lambda c, i: (c * NS + i, 0)),
        ],
        out_shape=[
            jax.ShapeDtypeStruct((2 * NROW, WROW), f32),
            jax.ShapeDtypeStruct((2 * NROW, 16), f32),
            jax.ShapeDtypeStruct((2 * NROW, WROW), f32),
        ],
    )(x, W_gat,
      jnp.stack([Asrc[:HH, :4], Asrc[HH:, 4:]]),
      jnp.stack([Adst[:HH, :4], Adst[HH:, 4:]]),
      E4)

    src = edge_index[0]
    dst = edge_index[1]
    pad = EPAD - E
    srcp = jnp.concatenate([src, jnp.zeros((pad,), jnp.int32)])
    dstp = jnp.concatenate([dst, jnp.full((pad,), N, jnp.int32)])

    acc = _edge_pass(srcp, dstp, D16, h2, acc0)

    P = jnp.concatenate([jnp.eye(HH, dtype=f32), jnp.zeros((16, HH), f32)],
                        axis=0)
    G = jnp.concatenate([jnp.zeros((HH, HH), f32), E4,
                         jnp.zeros((12, HH), f32)], axis=0)
    z, z1, z2 = pl.pallas_call(
        _post_body,
        grid=(NS,),
        in_specs=[
            pl.BlockSpec((ROWS_PT, WROW), lambda i: (i, 0)),
            pl.BlockSpec((ROWS_PT, WROW), lambda i: (NS + i, 0)),
            pl.BlockSpec((WROW, HH), lambda i: (0, 0)),
            pl.BlockSpec((WROW, HH), lambda i: (0, 0)),
            pl.BlockSpec((1, HID), lambda i: (0, 0)),
            pl.BlockSpec((HID, OUT), lambda i: (0, 0)),
            pl.BlockSpec((1, OUT), lambda i: (0, 0)),
            pl.BlockSpec((HID, OUT), lambda i: (0, 0)),
            pl.BlockSpec((1, OUT), lambda i: (0, 0)),
        ],
        out_specs=[
            pl.BlockSpec((ROWS_PT, HID), lambda i: (i, 0)),
            pl.BlockSpec((ROWS_PT, OUT), lambda i: (i, 0)),
            pl.BlockSpec((ROWS_PT, OUT), lambda i: (i, 0)),
        ],
        out_shape=[
            jax.ShapeDtypeStruct((N, HID), f32),
            jax.ShapeDtypeStruct((N, OUT), f32),
            jax.ShapeDtypeStruct((N, OUT), f32),
        ],
    )(acc, acc, P, G, b_gat.reshape(1, HID), W1, b1.reshape(1, OUT),
      W2, b2.reshape(1, OUT))
    return (z, z1, z2)


# speedup vs baseline: 63.6529x; 1.1370x over previous
import functools

import jax
import jax.numpy as jnp
from jax import lax
from jax.experimental import pallas as pl
from jax.experimental.pallas import tpu as pltpu
from jax.experimental.pallas import tpu_sc as plsc

N = 10000
E = 320000
IN = 128
H = 8
C = 32
HID = H * C
OUT = 256
HH = HID // 2

NS = 16
K = 80
NCH = 252
EPT = NCH * K
EPAD = NS * EPT
ROWS_PT = 632
NROW = NS * ROWS_PT

BN = 1000


def _pre_body(x_ref, wg_ref, asrc_ref, adst_ref, e4_ref,
              h2_ref, d16_ref, acc0_ref):
    hh = jnp.dot(x_ref[...], wg_ref[...], preferred_element_type=jnp.float32)
    s4 = jnp.dot(hh, asrc_ref[0], preferred_element_type=jnp.float32)
    d4 = jnp.dot(hh, adst_ref[0], preferred_element_type=jnp.float32)
    ss = s4 + d4
    exs4 = jnp.exp(jnp.where(ss > 0, ss, 0.2 * ss))
    h2_ref[...] = jnp.concatenate([hh, s4, s4, s4, s4], axis=1)
    d16_ref[...] = jnp.concatenate([d4, d4, d4, d4], axis=1)
    acc0_ref[...] = jnp.concatenate(
        [hh * jnp.dot(exs4, e4_ref[...], preferred_element_type=jnp.float32),
         exs4, exs4, exs4, exs4], axis=1)


def _post_body(a0_ref, a1_ref, p_ref, g_ref, bg_ref,
               w1_ref, b1_ref, w2_ref, b2_ref, z_ref, z1_ref, z2_ref):
    a0 = a0_ref[...]
    a1 = a1_ref[...]
    p = p_ref[...]
    g = g_ref[...]
    den0 = jnp.dot(a0, g, preferred_element_type=jnp.float32)
    den1 = jnp.dot(a1, g, preferred_element_type=jnp.float32)
    m0 = jnp.dot(a0, p, preferred_element_type=jnp.float32)
    m1 = jnp.dot(a1, p, preferred_element_type=jnp.float32)
    z = jnp.concatenate(
        [m0 / (den0 + 1e-16), m1 / (den1 + 1e-16)],
        axis=1) + bg_ref[...]
    z_ref[...] = z
    z1_ref[...] = jnp.dot(z, w1_ref[...],
                          preferred_element_type=jnp.float32) + b1_ref[...]
    z2_ref[...] = jnp.dot(z, w2_ref[...],
                          preferred_element_type=jnp.float32) + b2_ref[...]


WROW = HH + 16


def _edge_body(src_hbm, dst_hbm, d_hbm, h2_hbm, acc0_hbm,
               acc_out,
               src_v, dst_v, idx2_v, idxd_v, dsc_v, drows_v, hrows_v,
               acc_sh, sem_is, sem_id, sem_d, sem_h, sem_s):
    c = lax.axis_index("c")
    s = lax.axis_index("s")
    coff = c * NROW
    r0 = s * ROWS_PT
    pltpu.sync_copy(acc0_hbm.at[pl.ds(coff + r0, ROWS_PT)],
                    acc_sh.at[pl.ds(r0, ROWS_PT)])
    plsc.subcore_barrier()

    base = s * EPT

    def issue_idx(ch, b):
        off = base + ch * K
        pltpu.async_copy(src_hbm.at[pl.ds(off, K)], src_v.at[b],
                         sem_is.at[b])
        pltpu.async_copy(dst_hbm.at[pl.ds(off, K)], dst_v.at[b],
                         sem_id.at[b])

    def wait_idx(b):
        pltpu.make_async_copy(src_hbm.at[pl.ds(0, K)], src_v.at[b],
                              sem_is.at[b]).wait()
        pltpu.make_async_copy(dst_hbm.at[pl.ds(0, K)], dst_v.at[b],
                              sem_id.at[b]).wait()

    def comp_idx(b):
        def f(i, carry):
            sl = pl.ds(i * 16, 16)
            idx2_v[b, sl] = src_v[b, sl] + coff
            idxd_v[b, sl] = dst_v[b, sl] + coff
            dsc_v[b, sl] = dst_v[b, sl]
            return carry
        lax.fori_loop(0, K // 16, f, 0)

    def issue_gathers(b):
        pltpu.async_copy(d_hbm.at[idxd_v.at[b]], drows_v.at[b], sem_d.at[b])
        pltpu.async_copy(h2_hbm.at[idx2_v.at[b]], hrows_v.at[b], sem_h.at[b])

    def wait_gathers(b):
        pltpu.make_async_copy(d_hbm.at[idxd_v.at[b]], drows_v.at[b],
                              sem_d.at[b]).wait()
        pltpu.make_async_copy(h2_hbm.at[idx2_v.at[b]], hrows_v.at[b],
                              sem_h.at[b]).wait()

    def issue_scatter(b):
        pltpu.async_copy(hrows_v.at[b], acc_sh.at[dsc_v.at[b]], sem_s.at[b],
                         add=True)

    def wait_scatter(b):
        pltpu.make_async_copy(hrows_v.at[b], acc_sh.at[dsc_v.at[b]],
                              sem_s.at[b]).wait()

    def compute(b):
        def edge_f(e, carry):
            sv = hrows_v[b, e, pl.ds(HH, 16)]
            dv = drows_v[b, e, pl.ds(0, 16)]
            al = sv + dv
            ex = jnp.exp(jnp.where(al > 0, al, 0.2 * al))
            for j in range(HH // 16):
                hrows_v[b, e, pl.ds(j * 16, 16)] = (
                    hrows_v[b, e, pl.ds(j * 16, 16)]
                    * jnp.full((16,), ex[j // 2]))
            hrows_v[b, e, pl.ds(HH, 16)] = ex
            return carry
        lax.fori_loop(0, K, edge_f, 0)

    def step(ch, b):
        b1 = (b + 1) % 3
        b2 = (b + 2) % 3

        @pl.when(ch < NCH - 1)
        def _():
            wait_idx(b1)

        @pl.when(ch >= 2)
        def _():
            wait_scatter(b1)

        @pl.when(ch < NCH - 1)
        def _():
            comp_idx(b1)
            issue_gathers(b1)

        @pl.when(ch < NCH - 2)
        def _():
            issue_idx(ch + 2, b2)

        wait_gathers(b)
        compute(b)
        issue_scatter(b)

    issue_idx(0, 0)
    issue_idx(1, 1)
    wait_idx(0)
    comp_idx(0)
    issue_gathers(0)

    def triple(g, carry):
        step(3 * g, 0)
        step(3 * g + 1, 1)
        step(3 * g + 2, 2)
        return carry
    lax.fori_loop(0, NCH // 3, triple, 0)
    wait_scatter((NCH - 2) % 3)
    wait_scatter((NCH - 1) % 3)

    plsc.subcore_barrier()
    pltpu.sync_copy(acc_sh.at[pl.ds(r0, ROWS_PT)],
                    acc_out.at[pl.ds(coff + r0, ROWS_PT)])


def _edge_pass(srcp, dstp, D16, h2, acc0):
    mesh = plsc.VectorSubcoreMesh(core_axis_name="c", subcore_axis_name="s")
    f = functools.partial(
        pl.kernel,
        mesh=mesh,
        compiler_params=pltpu.CompilerParams(use_tc_tiling_on_sc=False),
        out_type=jax.ShapeDtypeStruct((2 * NROW, WROW), jnp.float32),
        scratch_types=[
            pltpu.VMEM((3, K), jnp.int32),
            pltpu.VMEM((3, K), jnp.int32),
            pltpu.VMEM((3, K), jnp.int32),
            pltpu.VMEM((3, K), jnp.int32),
            pltpu.VMEM((3, K), jnp.int32),
            pltpu.VMEM((3, K, 16), jnp.float32),
            pltpu.VMEM((3, K, WROW), jnp.float32),
            pltpu.VMEM_SHARED((NROW, WROW), jnp.float32),
            pltpu.SemaphoreType.DMA((3,)),
            pltpu.SemaphoreType.DMA((3,)),
            pltpu.SemaphoreType.DMA((3,)),
            pltpu.SemaphoreType.DMA((3,)),
            pltpu.SemaphoreType.DMA((3,)),
        ],
    )(_edge_body)
    return f(srcp, dstp, D16, h2, acc0)


def kernel(x, edge_index, W_gat, att_src, att_dst, b_gat, W1, b1, W2, b2):
    f32 = jnp.float32
    Asrc = (jnp.eye(H, dtype=f32)[:, None, :] * att_src[:, :, None]
            ).reshape(HID, H)
    Adst = (jnp.eye(H, dtype=f32)[:, None, :] * att_dst[:, :, None]
            ).reshape(HID, H)
    E4 = jnp.repeat(jnp.eye(4, dtype=f32), C, axis=1)

    h2, D16, acc0 = pl.pallas_call(
        _pre_body,
        grid=(2, NS),
        in_specs=[
            pl.BlockSpec((ROWS_PT, IN), lambda c, i: (i, 0)),
            pl.BlockSpec((IN, HH), lambda c, i: (0, c)),
            pl.BlockSpec((1, HH, 4), lambda c, i: (c, 0, 0)),
            pl.BlockSpec((1, HH, 4), lambda c, i: (c, 0, 0)),
            pl.BlockSpec((4, HH), lambda c, i: (0, 0)),
        ],
        out_specs=[
            pl.BlockSpec((ROWS_PT, WROW), lambda c, i: (c * NS + i, 0)),
            pl.BlockSpec((ROWS_PT, 16), lambda c, i: (c * NS + i, 0)),
            pl.BlockSpec((ROWS_PT, WROW), lambda c, i: (c * NS + i, 0)),
        ],
        out_shape=[
            jax.ShapeDtypeStruct((2 * NROW, WROW), f32),
            jax.ShapeDtypeStruct((2 * NROW, 16), f32),
            jax.ShapeDtypeStruct((2 * NROW, WROW), f32),
        ],
    )(x, W_gat,
      jnp.stack([Asrc[:HH, :4], Asrc[HH:, 4:]]),
      jnp.stack([Adst[:HH, :4], Adst[HH:, 4:]]),
      E4)

    src = edge_index[0]
    dst = edge_index[1]
    pad = EPAD - E
    srcp = jnp.concatenate([src, jnp.zeros((pad,), jnp.int32)])
    dstp = jnp.concatenate([dst, jnp.full((pad,), N, jnp.int32)])

    acc = _edge_pass(srcp, dstp, D16, h2, acc0)

    P = jnp.concatenate([jnp.eye(HH, dtype=f32), jnp.zeros((16, HH), f32)],
                        axis=0)
    G = jnp.concatenate([jnp.zeros((HH, HH), f32), E4,
                         jnp.zeros((12, HH), f32)], axis=0)
    z, z1, z2 = pl.pallas_call(
        _post_body,
        grid=(NS,),
        in_specs=[
            pl.BlockSpec((ROWS_PT, WROW), lambda i: (i, 0)),
            pl.BlockSpec((ROWS_PT, WROW), lambda i: (NS + i, 0)),
            pl.BlockSpec((WROW, HH), lambda i: (0, 0)),
            pl.BlockSpec((WROW, HH), lambda i: (0, 0)),
            pl.BlockSpec((1, HID), lambda i: (0, 0)),
            pl.BlockSpec((HID, OUT), lambda i: (0, 0)),
            pl.BlockSpec((1, OUT), lambda i: (0, 0)),
            pl.BlockSpec((HID, OUT), lambda i: (0, 0)),
            pl.BlockSpec((1, OUT), lambda i: (0, 0)),
        ],
        out_specs=[
            pl.BlockSpec((ROWS_PT, HID), lambda i: (i, 0)),
            pl.BlockSpec((ROWS_PT, OUT), lambda i: (i, 0)),
            pl.BlockSpec((ROWS_PT, OUT), lambda i: (i, 0)),
        ],
        out_shape=[
            jax.ShapeDtypeStruct((N, HID), f32),
            jax.ShapeDtypeStruct((N, OUT), f32),
            jax.ShapeDtypeStruct((N, OUT), f32),
        ],
    )(acc, acc, P, G, b_gat.reshape(1, HID), W1, b1.reshape(1, OUT),
      W2, b2.reshape(1, OUT))
    return (z, z1, z2)


# speedup vs baseline: 78.6209x; 1.2352x over previous
import functools

import jax
import jax.numpy as jnp
from jax import lax
from jax.experimental import pallas as pl
from jax.experimental.pallas import tpu as pltpu
from jax.experimental.pallas import tpu_sc as plsc

N = 10000
E = 320000
IN = 128
H = 8
C = 32
HID = H * C
OUT = 256
HH = HID // 2

NS = 16
K = 80
NCH = 252
EPT = NCH * K
EPAD = NS * EPT
ROWS_PT = 632
NROW = NS * ROWS_PT

BN = 1000


def _pre_body(x_ref, wg_ref, asrc_ref, adst_ref, e4_ref,
              h2_ref, d16_ref, acc0_ref):
    hh = jnp.dot(x_ref[...], wg_ref[...], preferred_element_type=jnp.float32)
    s4 = jnp.dot(hh, asrc_ref[0], preferred_element_type=jnp.float32)
    d4 = jnp.dot(hh, adst_ref[0], preferred_element_type=jnp.float32)
    ss = s4 + d4
    exs4 = jnp.exp(jnp.where(ss > 0, ss, 0.2 * ss))
    h2_ref[...] = jnp.concatenate([hh, s4, s4, s4, s4], axis=1)
    d16_ref[...] = jnp.concatenate([d4, d4, d4, d4], axis=1)
    acc0_ref[...] = jnp.concatenate(
        [hh * jnp.dot(exs4, e4_ref[...], preferred_element_type=jnp.float32),
         exs4, exs4, exs4, exs4], axis=1)


def _post_body(a0_ref, a1_ref, p_ref, g_ref, bg_ref,
               w1_ref, b1_ref, w2_ref, b2_ref, z_ref, z1_ref, z2_ref):
    a0 = a0_ref[...]
    a1 = a1_ref[...]
    p = p_ref[...]
    g = g_ref[...]
    den0 = jnp.dot(a0, g, preferred_element_type=jnp.float32)
    den1 = jnp.dot(a1, g, preferred_element_type=jnp.float32)
    m0 = jnp.dot(a0, p, preferred_element_type=jnp.float32)
    m1 = jnp.dot(a1, p, preferred_element_type=jnp.float32)
    z = jnp.concatenate(
        [m0 / (den0 + 1e-16), m1 / (den1 + 1e-16)],
        axis=1) + bg_ref[...]
    z_ref[...] = z
    z1_ref[...] = jnp.dot(z, w1_ref[...],
                          preferred_element_type=jnp.float32) + b1_ref[...]
    z2_ref[...] = jnp.dot(z, w2_ref[...],
                          preferred_element_type=jnp.float32) + b2_ref[...]


WROW = HH + 16


def _edge_body(src_hbm, dst_hbm, d_hbm, h2_hbm, acc0_hbm,
               acc_out,
               src_v, dst_v, idx2_v, idxd_v, dsc_v, drows_v, hrows_v,
               acc_sh, sem_is, sem_id, sem_d, sem_h, sem_s):
    c = lax.axis_index("c")
    s = lax.axis_index("s")
    coff = c * NROW
    r0 = s * ROWS_PT
    pltpu.sync_copy(acc0_hbm.at[pl.ds(coff + r0, ROWS_PT)],
                    acc_sh.at[pl.ds(r0, ROWS_PT)])
    plsc.subcore_barrier()

    base = s * EPT

    def issue_idx(ch, b):
        off = base + ch * K
        pltpu.async_copy(src_hbm.at[pl.ds(off, K)], src_v.at[b],
                         sem_is.at[b])
        pltpu.async_copy(dst_hbm.at[pl.ds(off, K)], dst_v.at[b],
                         sem_id.at[b])

    def wait_idx(b):
        pltpu.make_async_copy(src_hbm.at[pl.ds(0, K)], src_v.at[b],
                              sem_is.at[b]).wait()
        pltpu.make_async_copy(dst_hbm.at[pl.ds(0, K)], dst_v.at[b],
                              sem_id.at[b]).wait()

    def comp_idx(b):
        def f(i, carry):
            sl = pl.ds(i * 16, 16)
            idx2_v[b, sl] = src_v[b, sl] + coff
            idxd_v[b, sl] = dst_v[b, sl] + coff
            dsc_v[b, sl] = dst_v[b, sl]
            return carry
        lax.fori_loop(0, K // 16, f, 0)

    def issue_gathers(b):
        pltpu.async_copy(d_hbm.at[idxd_v.at[b]], drows_v.at[b], sem_d.at[b])
        pltpu.async_copy(h2_hbm.at[idx2_v.at[b]], hrows_v.at[b], sem_h.at[b])

    def wait_gathers(b):
        pltpu.make_async_copy(d_hbm.at[idxd_v.at[b]], drows_v.at[b],
                              sem_d.at[b]).wait()
        pltpu.make_async_copy(h2_hbm.at[idx2_v.at[b]], hrows_v.at[b],
                              sem_h.at[b]).wait()

    def issue_scatter(b):
        pltpu.async_copy(hrows_v.at[b], acc_sh.at[dsc_v.at[b]], sem_s.at[b],
                         add=True)

    def wait_scatter(b):
        pltpu.make_async_copy(hrows_v.at[b], acc_sh.at[dsc_v.at[b]],
                              sem_s.at[b]).wait()

    def compute(b):
        @plsc.parallel_loop(0, K, unroll=8)
        def _ex(e):
            sv = hrows_v[b, e, pl.ds(HH, 16)]
            dv = drows_v[b, e, pl.ds(0, 16)]
            al = sv + dv
            hrows_v[b, e, pl.ds(HH, 16)] = jnp.exp(jnp.maximum(al, 0.2 * al))

        @plsc.parallel_loop(0, K, unroll=2)
        def _scale(e):
            ex = hrows_v[b, e, pl.ds(HH, 16)]
            for j in range(HH // 16):
                hrows_v[b, e, pl.ds(j * 16, 16)] = (
                    hrows_v[b, e, pl.ds(j * 16, 16)]
                    * jnp.full((16,), ex[j // 2]))

    def step(ch, b):
        b1 = (b + 1) % 3
        b2 = (b + 2) % 3

        @pl.when(ch < NCH - 1)
        def _():
            wait_idx(b1)

        @pl.when(ch >= 2)
        def _():
            wait_scatter(b1)

        @pl.when(ch < NCH - 1)
        def _():
            comp_idx(b1)
            issue_gathers(b1)

        @pl.when(ch < NCH - 2)
        def _():
            issue_idx(ch + 2, b2)

        wait_gathers(b)
        compute(b)
        issue_scatter(b)

    issue_idx(0, 0)
    issue_idx(1, 1)
    wait_idx(0)
    comp_idx(0)
    issue_gathers(0)

    def triple(g, carry):
        step(3 * g, 0)
        step(3 * g + 1, 1)
        step(3 * g + 2, 2)
        return carry
    lax.fori_loop(0, NCH // 3, triple, 0)
    wait_scatter((NCH - 2) % 3)
    wait_scatter((NCH - 1) % 3)

    plsc.subcore_barrier()
    pltpu.sync_copy(acc_sh.at[pl.ds(r0, ROWS_PT)],
                    acc_out.at[pl.ds(coff + r0, ROWS_PT)])


def _edge_pass(srcp, dstp, D16, h2, acc0):
    mesh = plsc.VectorSubcoreMesh(core_axis_name="c", subcore_axis_name="s")
    f = functools.partial(
        pl.kernel,
        mesh=mesh,
        compiler_params=pltpu.CompilerParams(use_tc_tiling_on_sc=False),
        out_type=jax.ShapeDtypeStruct((2 * NROW, WROW), jnp.float32),
        scratch_types=[
            pltpu.VMEM((3, K), jnp.int32),
            pltpu.VMEM((3, K), jnp.int32),
            pltpu.VMEM((3, K), jnp.int32),
            pltpu.VMEM((3, K), jnp.int32),
            pltpu.VMEM((3, K), jnp.int32),
            pltpu.VMEM((3, K, 16), jnp.float32),
            pltpu.VMEM((3, K, WROW), jnp.float32),
            pltpu.VMEM_SHARED((NROW, WROW), jnp.float32),
            pltpu.SemaphoreType.DMA((3,)),
            pltpu.SemaphoreType.DMA((3,)),
            pltpu.SemaphoreType.DMA((3,)),
            pltpu.SemaphoreType.DMA((3,)),
            pltpu.SemaphoreType.DMA((3,)),
        ],
    )(_edge_body)
    return f(srcp, dstp, D16, h2, acc0)


def kernel(x, edge_index, W_gat, att_src, att_dst, b_gat, W1, b1, W2, b2):
    f32 = jnp.float32
    Asrc = (jnp.eye(H, dtype=f32)[:, None, :] * att_src[:, :, None]
            ).reshape(HID, H)
    Adst = (jnp.eye(H, dtype=f32)[:, None, :] * att_dst[:, :, None]
            ).reshape(HID, H)
    E4 = jnp.repeat(jnp.eye(4, dtype=f32), C, axis=1)

    h2, D16, acc0 = pl.pallas_call(
        _pre_body,
        grid=(2, NS),
        in_specs=[
            pl.BlockSpec((ROWS_PT, IN), lambda c, i: (i, 0)),
            pl.BlockSpec((IN, HH), lambda c, i: (0, c)),
            pl.BlockSpec((1, HH, 4), lambda c, i: (c, 0, 0)),
            pl.BlockSpec((1, HH, 4), lambda c, i: (c, 0, 0)),
            pl.BlockSpec((4, HH), lambda c, i: (0, 0)),
        ],
        out_specs=[
            pl.BlockSpec((ROWS_PT, WROW), lambda c, i: (c * NS + i, 0)),
            pl.BlockSpec((ROWS_PT, 16), lambda c, i: (c * NS + i, 0)),
            pl.BlockSpec((ROWS_PT, WROW), lambda c, i: (c * NS + i, 0)),
        ],
        out_shape=[
            jax.ShapeDtypeStruct((2 * NROW, WROW), f32),
            jax.ShapeDtypeStruct((2 * NROW, 16), f32),
            jax.ShapeDtypeStruct((2 * NROW, WROW), f32),
        ],
    )(x, W_gat,
      jnp.stack([Asrc[:HH, :4], Asrc[HH:, 4:]]),
      jnp.stack([Adst[:HH, :4], Adst[HH:, 4:]]),
      E4)

    src = edge_index[0]
    dst = edge_index[1]
    pad = EPAD - E
    srcp = jnp.concatenate([src, jnp.zeros((pad,), jnp.int32)])
    dstp = jnp.concatenate([dst, jnp.full((pad,), N, jnp.int32)])

    acc = _edge_pass(srcp, dstp, D16, h2, acc0)

    P = jnp.concatenate([jnp.eye(HH, dtype=f32), jnp.zeros((16, HH), f32)],
                        axis=0)
    G = jnp.concatenate([jnp.zeros((HH, HH), f32), E4,
                         jnp.zeros((12, HH), f32)], axis=0)
    z, z1, z2 = pl.pallas_call(
        _post_body,
        grid=(NS,),
        in_specs=[
            pl.BlockSpec((ROWS_PT, WROW), lambda i: (i, 0)),
            pl.BlockSpec((ROWS_PT, WROW), lambda i: (NS + i, 0)),
            pl.BlockSpec((WROW, HH), lambda i: (0, 0)),
            pl.BlockSpec((WROW, HH), lambda i: (0, 0)),
            pl.BlockSpec((1, HID), lambda i: (0, 0)),
            pl.BlockSpec((HID, OUT), lambda i: (0, 0)),
            pl.BlockSpec((1, OUT), lambda i: (0, 0)),
            pl.BlockSpec((HID, OUT), lambda i: (0, 0)),
            pl.BlockSpec((1, OUT), lambda i: (0, 0)),
        ],
        out_specs=[
            pl.BlockSpec((ROWS_PT, HID), lambda i: (i, 0)),
            pl.BlockSpec((ROWS_PT, OUT), lambda i: (i, 0)),
            pl.BlockSpec((ROWS_PT, OUT), lambda i: (i, 0)),
        ],
        out_shape=[
            jax.ShapeDtypeStruct((N, HID), f32),
            jax.ShapeDtypeStruct((N, OUT), f32),
            jax.ShapeDtypeStruct((N, OUT), f32),
        ],
    )(acc, acc, P, G, b_gat.reshape(1, HID), W1, b1.reshape(1, OUT),
      W2, b2.reshape(1, OUT))
    return (z, z1, z2)


# speedup vs baseline: 78.6603x; 1.0005x over previous
import functools

import jax
import jax.numpy as jnp
from jax import lax
from jax.experimental import pallas as pl
from jax.experimental.pallas import tpu as pltpu
from jax.experimental.pallas import tpu_sc as plsc

N = 10000
E = 320000
IN = 128
H = 8
C = 32
HID = H * C
OUT = 256
HH = HID // 2

NS = 16
K = 80
NCH = 252
EPT = NCH * K
EPAD = NS * EPT
ROWS_PT = 632
NROW = NS * ROWS_PT

BN = 1000


def _pre_body(x_ref, wg_ref, asrc_ref, adst_ref, e4_ref,
              h2_ref, d16_ref, acc0_ref):
    hh = jnp.dot(x_ref[...], wg_ref[...], preferred_element_type=jnp.float32)
    s4 = jnp.dot(hh, asrc_ref[0], preferred_element_type=jnp.float32)
    d4 = jnp.dot(hh, adst_ref[0], preferred_element_type=jnp.float32)
    ss = s4 + d4
    exs4 = jnp.exp(jnp.where(ss > 0, ss, 0.2 * ss))
    h2_ref[...] = jnp.concatenate([hh, s4, s4, s4, s4], axis=1)
    d16_ref[...] = jnp.concatenate([d4, d4, d4, d4], axis=1)
    acc0_ref[...] = jnp.concatenate(
        [hh * jnp.dot(exs4, e4_ref[...], preferred_element_type=jnp.float32),
         exs4, exs4, exs4, exs4], axis=1)


def _post_body(a0_ref, a1_ref, p_ref, g_ref, bg_ref,
               w1_ref, b1_ref, w2_ref, b2_ref, z_ref, z1_ref, z2_ref):
    a0 = a0_ref[...]
    a1 = a1_ref[...]
    p = p_ref[...]
    g = g_ref[...]
    den0 = jnp.dot(a0, g, preferred_element_type=jnp.float32)
    den1 = jnp.dot(a1, g, preferred_element_type=jnp.float32)
    m0 = jnp.dot(a0, p, preferred_element_type=jnp.float32)
    m1 = jnp.dot(a1, p, preferred_element_type=jnp.float32)
    z = jnp.concatenate(
        [m0 / (den0 + 1e-16), m1 / (den1 + 1e-16)],
        axis=1) + bg_ref[...]
    z_ref[...] = z
    z1_ref[...] = jnp.dot(z, w1_ref[...],
                          preferred_element_type=jnp.float32) + b1_ref[...]
    z2_ref[...] = jnp.dot(z, w2_ref[...],
                          preferred_element_type=jnp.float32) + b2_ref[...]


WROW = HH + 16


def _edge_body(src_hbm, dst_hbm, d_hbm, h2_hbm, acc0_hbm,
               acc_out,
               src_v, dst_v, idx2_v, idxd_v, dsc_v, drows_v, hrows_v,
               acc_sh, sem_is, sem_id, sem_d, sem_h, sem_s):
    c = lax.axis_index("c")
    s = lax.axis_index("s")
    coff = c * NROW
    r0 = s * ROWS_PT
    pltpu.sync_copy(acc0_hbm.at[pl.ds(coff + r0, ROWS_PT)],
                    acc_sh.at[pl.ds(r0, ROWS_PT)])
    plsc.subcore_barrier()

    base = s * EPT

    def issue_idx(ch, b):
        off = base + ch * K
        pltpu.async_copy(src_hbm.at[pl.ds(off, K)], src_v.at[b],
                         sem_is.at[b])
        pltpu.async_copy(dst_hbm.at[pl.ds(off, K)], dst_v.at[b],
                         sem_id.at[b])

    def wait_idx(b):
        pltpu.make_async_copy(src_hbm.at[pl.ds(0, K)], src_v.at[b],
                              sem_is.at[b]).wait()
        pltpu.make_async_copy(dst_hbm.at[pl.ds(0, K)], dst_v.at[b],
                              sem_id.at[b]).wait()

    def comp_idx(b):
        @plsc.parallel_loop(0, K // 16, unroll=5)
        def _f(i):
            sl = pl.ds(i * 16, 16)
            idx2_v[b, sl] = src_v[b, sl] + coff
            idxd_v[b, sl] = dst_v[b, sl] + coff
            dsc_v[b, sl] = dst_v[b, sl]

    def issue_gathers(b):
        pltpu.async_copy(d_hbm.at[idxd_v.at[b]], drows_v.at[b], sem_d.at[b])
        pltpu.async_copy(h2_hbm.at[idx2_v.at[b]], hrows_v.at[b], sem_h.at[b])

    def wait_gathers(b):
        pltpu.make_async_copy(d_hbm.at[idxd_v.at[b]], drows_v.at[b],
                              sem_d.at[b]).wait()
        pltpu.make_async_copy(h2_hbm.at[idx2_v.at[b]], hrows_v.at[b],
                              sem_h.at[b]).wait()

    def issue_scatter(b):
        pltpu.async_copy(hrows_v.at[b], acc_sh.at[dsc_v.at[b]], sem_s.at[b],
                         add=True)

    def wait_scatter(b):
        pltpu.make_async_copy(hrows_v.at[b], acc_sh.at[dsc_v.at[b]],
                              sem_s.at[b]).wait()

    def compute(b):
        @plsc.parallel_loop(0, K, unroll=8)
        def _ex(e):
            sv = hrows_v[b, e, pl.ds(HH, 16)]
            dv = drows_v[b, e, pl.ds(0, 16)]
            al = sv + dv
            hrows_v[b, e, pl.ds(HH, 16)] = jnp.exp(jnp.maximum(al, 0.2 * al))

        @plsc.parallel_loop(0, K, unroll=4)
        def _scale(e):
            ex = hrows_v[b, e, pl.ds(HH, 16)]
            for j in range(HH // 16):
                hrows_v[b, e, pl.ds(j * 16, 16)] = (
                    hrows_v[b, e, pl.ds(j * 16, 16)]
                    * jnp.full((16,), ex[j // 2]))

    def step(ch, b):
        b1 = (b + 1) % 3
        b2 = (b + 2) % 3

        @pl.when(ch < NCH - 1)
        def _():
            wait_idx(b1)

        @pl.when(ch >= 2)
        def _():
            wait_scatter(b1)

        @pl.when(ch < NCH - 1)
        def _():
            comp_idx(b1)
            issue_gathers(b1)

        @pl.when(ch < NCH - 2)
        def _():
            issue_idx(ch + 2, b2)

        wait_gathers(b)
        compute(b)
        issue_scatter(b)

    issue_idx(0, 0)
    issue_idx(1, 1)
    wait_idx(0)
    comp_idx(0)
    issue_gathers(0)

    def triple(g, carry):
        step(3 * g, 0)
        step(3 * g + 1, 1)
        step(3 * g + 2, 2)
        return carry
    lax.fori_loop(0, NCH // 3, triple, 0)
    wait_scatter((NCH - 2) % 3)
    wait_scatter((NCH - 1) % 3)

    plsc.subcore_barrier()
    pltpu.sync_copy(acc_sh.at[pl.ds(r0, ROWS_PT)],
                    acc_out.at[pl.ds(coff + r0, ROWS_PT)])


def _edge_pass(srcp, dstp, D16, h2, acc0):
    mesh = plsc.VectorSubcoreMesh(core_axis_name="c", subcore_axis_name="s")
    f = functools.partial(
        pl.kernel,
        mesh=mesh,
        compiler_params=pltpu.CompilerParams(use_tc_tiling_on_sc=False),
        out_type=jax.ShapeDtypeStruct((2 * NROW, WROW), jnp.float32),
        scratch_types=[
            pltpu.VMEM((3, K), jnp.int32),
            pltpu.VMEM((3, K), jnp.int32),
            pltpu.VMEM((3, K), jnp.int32),
            pltpu.VMEM((3, K), jnp.int32),
            pltpu.VMEM((3, K), jnp.int32),
            pltpu.VMEM((3, K, 16), jnp.float32),
            pltpu.VMEM((3, K, WROW), jnp.float32),
            pltpu.VMEM_SHARED((NROW, WROW), jnp.float32),
            pltpu.SemaphoreType.DMA((3,)),
            pltpu.SemaphoreType.DMA((3,)),
            pltpu.SemaphoreType.DMA((3,)),
            pltpu.SemaphoreType.DMA((3,)),
            pltpu.SemaphoreType.DMA((3,)),
        ],
    )(_edge_body)
    return f(srcp, dstp, D16, h2, acc0)


def kernel(x, edge_index, W_gat, att_src, att_dst, b_gat, W1, b1, W2, b2):
    f32 = jnp.float32
    Asrc = (jnp.eye(H, dtype=f32)[:, None, :] * att_src[:, :, None]
            ).reshape(HID, H)
    Adst = (jnp.eye(H, dtype=f32)[:, None, :] * att_dst[:, :, None]
            ).reshape(HID, H)
    E4 = jnp.repeat(jnp.eye(4, dtype=f32), C, axis=1)

    h2, D16, acc0 = pl.pallas_call(
        _pre_body,
        grid=(2, NS),
        in_specs=[
            pl.BlockSpec((ROWS_PT, IN), lambda c, i: (i, 0)),
            pl.BlockSpec((IN, HH), lambda c, i: (0, c)),
            pl.BlockSpec((1, HH, 4), lambda c, i: (c, 0, 0)),
            pl.BlockSpec((1, HH, 4), lambda c, i: (c, 0, 0)),
            pl.BlockSpec((4, HH), lambda c, i: (0, 0)),
        ],
        out_specs=[
            pl.BlockSpec((ROWS_PT, WROW), lambda c, i: (c * NS + i, 0)),
            pl.BlockSpec((ROWS_PT, 16), lambda c, i: (c * NS + i, 0)),
            pl.BlockSpec((ROWS_PT, WROW), lambda c, i: (c * NS + i, 0)),
        ],
        out_shape=[
            jax.ShapeDtypeStruct((2 * NROW, WROW), f32),
            jax.ShapeDtypeStruct((2 * NROW, 16), f32),
            jax.ShapeDtypeStruct((2 * NROW, WROW), f32),
        ],
    )(x, W_gat,
      jnp.stack([Asrc[:HH, :4], Asrc[HH:, 4:]]),
      jnp.stack([Adst[:HH, :4], Adst[HH:, 4:]]),
      E4)

    src = edge_index[0]
    dst = edge_index[1]
    pad = EPAD - E
    srcp = jnp.concatenate([src, jnp.zeros((pad,), jnp.int32)])
    dstp = jnp.concatenate([dst, jnp.full((pad,), N, jnp.int32)])

    acc = _edge_pass(srcp, dstp, D16, h2, acc0)

    P = jnp.concatenate([jnp.eye(HH, dtype=f32), jnp.zeros((16, HH), f32)],
                        axis=0)
    G = jnp.concatenate([jnp.zeros((HH, HH), f32), E4,
                         jnp.zeros((12, HH), f32)], axis=0)
    z, z1, z2 = pl.pallas_call(
        _post_body,
        grid=(NS,),
        in_specs=[
            pl.BlockSpec((ROWS_PT, WROW), lambda i: (i, 0)),
            pl.BlockSpec((ROWS_PT, WROW), lambda i: (NS + i, 0)),
            pl.BlockSpec((WROW, HH), lambda i: (0, 0)),
            pl.BlockSpec((WROW, HH), lambda i: (0, 0)),
            pl.BlockSpec((1, HID), lambda i: (0, 0)),
            pl.BlockSpec((HID, OUT), lambda i: (0, 0)),
            pl.BlockSpec((1, OUT), lambda i: (0, 0)),
            pl.BlockSpec((HID, OUT), lambda i: (0, 0)),
            pl.BlockSpec((1, OUT), lambda i: (0, 0)),
        ],
        out_specs=[
            pl.BlockSpec((ROWS_PT, HID), lambda i: (i, 0)),
            pl.BlockSpec((ROWS_PT, OUT), lambda i: (i, 0)),
            pl.BlockSpec((ROWS_PT, OUT), lambda i: (i, 0)),
        ],
        out_shape=[
            jax.ShapeDtypeStruct((N, HID), f32),
            jax.ShapeDtypeStruct((N, OUT), f32),
            jax.ShapeDtypeStruct((N, OUT), f32),
        ],
    )(acc, acc, P, G, b_gat.reshape(1, HID), W1, b1.reshape(1, OUT),
      W2, b2.reshape(1, OUT))
    return (z, z1, z2)
